# baseline (device time: 33861 ns/iter reference)
import os

import jax
import jax.numpy as jnp
from jax import lax
from jax.experimental import pallas as pl
from jax.experimental.pallas import tpu as pltpu

try:
    _ABLATE = open(
        os.path.join(os.path.dirname(__file__), "ablate.txt")
    ).read().strip()
except OSError:
    _ABLATE = ""

N_DEV = 16
_NO_EXPLICIT_BARRIER = "nobarrier" in _ABLATE
PACK = 128
CHUNK_B = 2
NCHUNK = 4


def kernel(Q, K, V):
    b, sq, h, d = Q.shape
    kv = K.shape[1]
    hd = h * d
    rows = CHUNK_B * h

    Q2 = Q.reshape(b, hd)
    K2 = K.reshape(b, kv, hd)
    V2 = V.reshape(b, kv, hd)

    def body(q_ref, k_hbm, v_hbm, out_ref,
             kbuf, vbuf, gather_ref,
             kcp_sems, vcp_sems, send_sems, recv_sems):
        my_pos = lax.axis_index("i")
        scale = d ** -0.5

        col_h = lax.broadcasted_iota(jnp.int32, (h, hd), 1) // d
        row_h = lax.broadcasted_iota(jnp.int32, (h, hd), 0)
        hmask = col_h == row_h

        def start_chunk(c, slot):
            kcp = pltpu.make_async_copy(
                k_hbm.at[pl.ds(c * CHUNK_B, CHUNK_B)], kbuf.at[slot],
                kcp_sems.at[slot])
            vcp = pltpu.make_async_copy(
                v_hbm.at[pl.ds(c * CHUNK_B, CHUNK_B)], vbuf.at[slot],
                vcp_sems.at[slot])
            kcp.start()
            vcp.start()
            return kcp, vcp

        copies = [None, None]
        copies[0] = start_chunk(0, 0)

        if not _NO_EXPLICIT_BARRIER:
            barrier_sem = pltpu.get_barrier_semaphore()
            for off in range(1, N_DEV):
                pl.semaphore_signal(
                    barrier_sem, inc=1,
                    device_id=(lax.rem(my_pos + off, N_DEV),),
                    device_id_type=pl.DeviceIdType.MESH,
                )
            pl.semaphore_wait(barrier_sem, N_DEV - 1)

        sends = []
        for c in range(NCHUNK):
            slot = c % 2
            if c + 1 < NCHUNK:
                copies[1 - slot] = start_chunk(c + 1, 1 - slot)
            kcp, vcp = copies[slot]
            kcp.wait()
            vcp.wait()

            for cb in range(CHUNK_B):
                bb = c * CHUNK_B + cb
                qflat = q_ref[pl.ds(bb, 1), :]
                qmask_t = jnp.where(hmask, qflat, 0.0)
                k_b = kbuf[slot, cb]
                v_b = vbuf[slot, cb]
                s_t = lax.dot_general(
                    qmask_t, k_b,
                    dimension_numbers=(((1,), (1,)), ((), ())),
                    preferred_element_type=jnp.float32,
                ) * scale
                m = jnp.max(s_t, axis=1, keepdims=True)
                p = jnp.exp(s_t - m)
                l = jnp.sum(p, axis=1, keepdims=True)
                o_full = lax.dot_general(
                    p, v_b,
                    dimension_numbers=(((1,), (0,)), ((), ())),
                    preferred_element_type=jnp.float32,
                )
                o = jnp.concatenate(
                    [o_full[hh:hh + 1, hh * d:(hh + 1) * d]
                     for hh in range(h)], axis=0)
                gather_ref[my_pos, c, pl.ds(cb * h, h), pl.ds(0, d)] = o
                gather_ref[my_pos, c, pl.ds(cb * h, h), pl.ds(d, 1)] = m
                gather_ref[my_pos, c, pl.ds(cb * h, h), pl.ds(d + 1, 1)] = l

            for off in range(1, N_DEV):
                dst = lax.rem(my_pos + off, N_DEV)
                rdma = pltpu.make_async_remote_copy(
                    src_ref=gather_ref.at[my_pos, c],
                    dst_ref=gather_ref.at[my_pos, c],
                    send_sem=send_sems.at[off, c],
                    recv_sem=recv_sems.at[off, c],
                    device_id=(dst,),
                    device_id_type=pl.DeviceIdType.MESH,
                )
                rdma.start()
                sends.append(rdma)

        for rdma in sends:
            rdma.wait_send()
        for rdma in sends:
            rdma.wait_recv()

        slots = [gather_ref[s].reshape(b * h, PACK) for s in range(N_DEV)]
        gm = slots[0][:, d:d + 1]
        for s_idx in range(1, N_DEV):
            gm = jnp.maximum(gm, slots[s_idx][:, d:d + 1])
        o_tot = jnp.zeros((b * h, d), jnp.float32)
        l_tot = jnp.zeros((b * h, 1), jnp.float32)
        for s_idx in range(N_DEV):
            w = jnp.exp(slots[s_idx][:, d:d + 1] - gm)
            l_tot = l_tot + w * slots[s_idx][:, d + 1:d + 2]
            o_tot = o_tot + w * slots[s_idx][:, 0:d]
        out = o_tot / l_tot
        out_ref[:, 0, :, :] = out.reshape(b, h, d)

    return pl.pallas_call(
        body,
        out_shape=jax.ShapeDtypeStruct((b, sq, h, d), jnp.float32),
        in_specs=[
            pl.BlockSpec(memory_space=pltpu.VMEM),
            pl.BlockSpec(memory_space=pltpu.MemorySpace.HBM),
            pl.BlockSpec(memory_space=pltpu.MemorySpace.HBM),
        ],
        out_specs=pl.BlockSpec(memory_space=pltpu.VMEM),
        scratch_shapes=[
            pltpu.VMEM((2, CHUNK_B, kv, hd), jnp.float32),
            pltpu.VMEM((2, CHUNK_B, kv, hd), jnp.float32),
            pltpu.VMEM((N_DEV, NCHUNK, rows, PACK), jnp.float32),
            pltpu.SemaphoreType.DMA((2,)),
            pltpu.SemaphoreType.DMA((2,)),
            pltpu.SemaphoreType.DMA((N_DEV, NCHUNK)),
            pltpu.SemaphoreType.DMA((N_DEV, NCHUNK)),
        ],
        **(
            {}
            if _NO_EXPLICIT_BARRIER
            else {"compiler_params": pltpu.CompilerParams(collective_id=0)}
        ),
    )(Q2, K2, V2)
